# baseline (speedup 1.0000x reference)
import functools

import jax
import jax.numpy as jnp
from jax import lax
from jax.experimental import pallas as pl
from jax.experimental.pallas import tpu as pltpu
from jax.experimental.pallas import tpu_sc as plsc


TT = 256
TM = 256


def _qkv_kernel(x_ref, g_ref, w_ref, out_ref):
    xv = x_ref[...]
    h = xv * jax.lax.rsqrt(jnp.mean(xv * xv, axis=-1, keepdims=True) + 1e-6)
    h = (h * g_ref[...]).astype(jnp.bfloat16)
    out_ref[...] = jnp.dot(h, w_ref[...],
                           preferred_element_type=jnp.float32
                           ).astype(jnp.bfloat16)


def _attn_kernel(q_ref, k_ref, v_ref, o_ref, *, scale, n_heads, dk):
    i = pl.program_id(0)
    q = q_ref[...]
    nq = q.shape[0]
    row = jax.lax.broadcasted_iota(jnp.int32, (nq, nq), 0)
    col = jax.lax.broadcasted_iota(jnp.int32, (nq, nq), 1)
    diag_bias = jnp.where(col <= row, 0.0, -1e30)
    outs = []
    for h in range(n_heads):
        qh = q[:, h * dk:(h + 1) * dk]

        def body(j, carry, h=h, qh=qh):
            m0, l0, acc0 = carry
            kh = k_ref[pl.ds(j * nq, nq), h * dk:(h + 1) * dk]
            vh = v_ref[pl.ds(j * nq, nq), h * dk:(h + 1) * dk]
            s = jax.lax.dot_general(qh, kh, (((1,), (1,)), ((), ())),
                                    preferred_element_type=jnp.float32)
            s = s * scale + jnp.where(j == i, 1.0, 0.0) * diag_bias
            m1 = jnp.maximum(m0, jnp.max(s, axis=-1, keepdims=True))
            p = jnp.exp(s - m1)
            corr = jnp.exp(m0 - m1)
            l1 = l0 * corr + jnp.sum(p, axis=-1, keepdims=True)
            acc1 = acc0 * corr + jnp.dot(p.astype(jnp.bfloat16), vh,
                                         preferred_element_type=jnp.float32)
            return m1, l1, acc1

        m, l, acc = jax.lax.fori_loop(
            0, i + 1, body,
            (jnp.full((nq, 1), -1e30, jnp.float32),
             jnp.zeros((nq, 1), jnp.float32),
             jnp.zeros((nq, dk), jnp.float32)))
        outs.append((acc / l).astype(jnp.bfloat16))
    o_ref[...] = jnp.concatenate(outs, axis=1)


def _proj_router_kernel(x_ref, o_ref, w_ref, b_ref, g_ref, wg_ref, bg_ref,
                        wv_ref, bv_ref, n_ref,
                        x2_ref, h2_ref, coef_ref, tm_ref):
    x2 = (x_ref[...]
          + jnp.dot(o_ref[...], w_ref[...],
                    preferred_element_type=jnp.float32)
          + b_ref[...])
    x2_ref[...] = x2
    h2 = x2 * jax.lax.rsqrt(jnp.mean(x2 * x2, axis=-1, keepdims=True) + 1e-6)
    h2 = h2 * g_ref[...]
    h2_ref[...] = h2
    lg = jnp.dot(h2, wg_ref[...], preferred_element_type=jnp.float32) + bg_ref[...]
    lv = jnp.dot(h2, wv_ref[...], preferred_element_type=jnp.float32) + bv_ref[...]
    sp = jnp.maximum(lv, 0.0) + jnp.log(1.0 + jnp.exp(-jnp.abs(lv)))
    logits = lg + n_ref[...] * sp
    m1 = jnp.max(logits, axis=-1, keepdims=True)
    neg = jnp.where(logits == m1, -jnp.inf, logits)
    m2 = jnp.max(neg, axis=-1, keepdims=True)
    tmask = logits >= m2
    z = jnp.where(tmask, jnp.exp(logits - m1), 0.0)
    coef_ref[...] = z / jnp.sum(z, axis=-1, keepdims=True)
    tm_ref[...] = tmask.astype(jnp.float32)


def _offsets_kernel(tm_ref, te_ref, offp_ref, *, n_experts, n_tiles, tile):
    cnt = jnp.sum(tm_ref[...], axis=0, keepdims=True)
    ntile = jnp.ceil(cnt / tile)
    e = n_experts
    mT = (jax.lax.broadcasted_iota(jnp.int32, (e, e), 0)
          < jax.lax.broadcasted_iota(jnp.int32, (e, e), 1)).astype(jnp.float32)
    toff = jnp.dot(ntile, mT, preferred_element_type=jnp.float32)
    offp_ref[...] = toff * tile
    m = (jax.lax.broadcasted_iota(jnp.int32, (e, e), 1)
         < jax.lax.broadcasted_iota(jnp.int32, (e, e), 0)).astype(jnp.float32)
    ntile_b = jnp.broadcast_to(ntile, (e, e))
    toff_s = jnp.sum(m * ntile_b, axis=1, keepdims=True)
    pio = jax.lax.broadcasted_iota(
        jnp.int32, (e, n_tiles), 1).astype(jnp.float32)
    ind = (toff_s <= pio).astype(jnp.int32)
    te = jnp.sum(ind, axis=0, keepdims=True) - 1
    te_ref[...] = jnp.clip(te, 0, n_experts - 1)


def _slots_kernel(tmf_ref, coef_ref, tm_ref, offp_ref,
                  s1_ref, s2_ref, c1_ref, c2_ref, *, n_slots):
    i = pl.program_id(0)
    tt, e = coef_ref.shape
    t_all = tmf_ref.shape[0]
    rowg = i * tt + jax.lax.broadcasted_iota(jnp.int32, (tt, t_all), 0)
    colg = jax.lax.broadcasted_iota(jnp.int32, (tt, t_all), 1)
    ltri = (colg < rowg).astype(jnp.float32)
    rank = jnp.dot(ltri, tmf_ref[...], preferred_element_type=jnp.float32)
    eiota = jax.lax.broadcasted_iota(jnp.int32, (tt, e), 1)
    sel = tm_ref[...] > 0.5
    e1 = jnp.min(jnp.where(sel, eiota, 99), axis=-1, keepdims=True)
    e2 = jnp.max(jnp.where(sel, eiota, -1), axis=-1, keepdims=True)
    oh1 = eiota == e1
    oh2 = eiota == e2
    coef = coef_ref[...]
    offb = jnp.broadcast_to(offp_ref[...], (tt, e))
    c1_ref[...] = jnp.sum(jnp.where(oh1, coef, 0.0), axis=-1, keepdims=True)
    c2_ref[...] = jnp.sum(jnp.where(oh2, coef, 0.0), axis=-1, keepdims=True)
    s1 = jnp.sum(jnp.where(oh1, offb + rank, 0.0), axis=-1, keepdims=True)
    s2 = jnp.sum(jnp.where(oh2, offb + rank, 0.0), axis=-1, keepdims=True)
    s1_ref[...] = jnp.clip(s1.astype(jnp.int32), 0, n_slots - 1)
    s2_ref[...] = jnp.clip(s2.astype(jnp.int32), 0, n_slots - 1)


def _group_ffn_kernel(te_ref, x_ref, w1_ref, b1_ref, w2_ref, b2_ref, y_ref):
    xb = x_ref[...].astype(jnp.bfloat16)
    hm = jnp.maximum(
        jnp.dot(xb, w1_ref[0].astype(jnp.bfloat16),
                preferred_element_type=jnp.float32) + b1_ref[0], 0.0)
    y_ref[...] = (jnp.dot(hm.astype(jnp.bfloat16),
                          w2_ref[0].astype(jnp.bfloat16),
                          preferred_element_type=jnp.float32)
                  + b2_ref[0])


def _combine_kernel(x2_ref, y1_ref, y2_ref, c1_ref, c2_ref, out_ref):
    out_ref[...] = (x2_ref[...]
                    + c1_ref[...] * y1_ref[...]
                    + c2_ref[...] * y2_ref[...])


def kernel(x, noise, g1, g2, Wq, Wk, Wv, Wo, bo, Wg, bg, Wvar, bvar,
           W1, b1, W2, b2):
    B, T, D = x.shape
    H, _, DK = Wq.shape
    E = Wg.shape[1]
    DFF = W1.shape[2]
    HD = H * DK
    tt = min(TT, T)
    nt = T // tt
    K = 2
    NTILES = (T * K) // TM + E
    S = NTILES * TM

    x2d = x.reshape(T, D)
    n2d = noise.reshape(T, E)
    g1r = g1.reshape(1, D)
    g2r = g2.reshape(1, D)
    bor = bo.reshape(1, D)
    bgr = bg.reshape(1, E)
    bvr = bvar.reshape(1, E)
    Wqkv = jnp.concatenate(
        [jnp.transpose(w, (1, 0, 2)).reshape(D, HD) for w in (Wq, Wk, Wv)],
        axis=1).astype(jnp.bfloat16)
    Wob = Wo.astype(jnp.bfloat16)

    qkv = pl.pallas_call(
        _qkv_kernel,
        grid=(nt,),
        in_specs=[
            pl.BlockSpec((tt, D), lambda i: (i, 0)),
            pl.BlockSpec((1, D), lambda i: (0, 0)),
            pl.BlockSpec((D, 3 * HD), lambda i: (0, 0)),
        ],
        out_specs=pl.BlockSpec((tt, 3 * HD), lambda i: (i, 0)),
        out_shape=jax.ShapeDtypeStruct((T, 3 * HD), jnp.bfloat16),
    )(x2d, g1r, Wqkv)

    o2 = pl.pallas_call(
        functools.partial(_attn_kernel, scale=1.0 / (DK ** 0.5),
                          n_heads=H, dk=DK),
        grid=(nt,),
        in_specs=[
            pl.BlockSpec((tt, HD), lambda i: (i, 0)),
            pl.BlockSpec((T, HD), lambda i: (0, 1)),
            pl.BlockSpec((T, HD), lambda i: (0, 2)),
        ],
        out_specs=pl.BlockSpec((tt, HD), lambda i: (i, 0)),
        out_shape=jax.ShapeDtypeStruct((T, HD), jnp.bfloat16),
    )(qkv, qkv, qkv)

    x2, h2, coef, tm = pl.pallas_call(
        _proj_router_kernel,
        grid=(nt,),
        in_specs=[
            pl.BlockSpec((tt, D), lambda i: (i, 0)),
            pl.BlockSpec((tt, HD), lambda i: (i, 0)),
            pl.BlockSpec((D, D), lambda i: (0, 0)),
            pl.BlockSpec((1, D), lambda i: (0, 0)),
            pl.BlockSpec((1, D), lambda i: (0, 0)),
            pl.BlockSpec((D, E), lambda i: (0, 0)),
            pl.BlockSpec((1, E), lambda i: (0, 0)),
            pl.BlockSpec((D, E), lambda i: (0, 0)),
            pl.BlockSpec((1, E), lambda i: (0, 0)),
            pl.BlockSpec((tt, E), lambda i: (i, 0)),
        ],
        out_specs=[
            pl.BlockSpec((tt, D), lambda i: (i, 0)),
            pl.BlockSpec((tt, D), lambda i: (i, 0)),
            pl.BlockSpec((tt, E), lambda i: (i, 0)),
            pl.BlockSpec((tt, E), lambda i: (i, 0)),
        ],
        out_shape=[
            jax.ShapeDtypeStruct((T, D), jnp.float32),
            jax.ShapeDtypeStruct((T, D), jnp.float32),
            jax.ShapeDtypeStruct((T, E), jnp.float32),
            jax.ShapeDtypeStruct((T, E), jnp.float32),
        ],
    )(x2d, o2, Wob, bor, g2r, Wg, bgr, Wvar, bvr, n2d)

    te2d, offp = pl.pallas_call(
        functools.partial(_offsets_kernel, n_experts=E, n_tiles=NTILES,
                          tile=TM),
        grid=(1,),
        in_specs=[pl.BlockSpec((T, E), lambda i: (0, 0))],
        out_specs=[
            pl.BlockSpec((1, NTILES), lambda i: (0, 0)),
            pl.BlockSpec((1, E), lambda i: (0, 0)),
        ],
        out_shape=[
            jax.ShapeDtypeStruct((1, NTILES), jnp.int32),
            jax.ShapeDtypeStruct((1, E), jnp.float32),
        ],
    )(tm)

    s1, s2, c1, c2 = pl.pallas_call(
        functools.partial(_slots_kernel, n_slots=S),
        grid=(nt,),
        in_specs=[
            pl.BlockSpec((T, E), lambda i: (0, 0)),
            pl.BlockSpec((tt, E), lambda i: (i, 0)),
            pl.BlockSpec((tt, E), lambda i: (i, 0)),
            pl.BlockSpec((1, E), lambda i: (0, 0)),
        ],
        out_specs=[
            pl.BlockSpec((tt, 1), lambda i: (i, 0)),
            pl.BlockSpec((tt, 1), lambda i: (i, 0)),
            pl.BlockSpec((tt, 1), lambda i: (i, 0)),
            pl.BlockSpec((tt, 1), lambda i: (i, 0)),
        ],
        out_shape=[
            jax.ShapeDtypeStruct((T, 1), jnp.int32),
            jax.ShapeDtypeStruct((T, 1), jnp.int32),
            jax.ShapeDtypeStruct((T, 1), jnp.float32),
            jax.ShapeDtypeStruct((T, 1), jnp.float32),
        ],
    )(tm, coef, tm, offp)

    info = plsc.get_sparse_core_info()
    NW = info.num_cores * info.num_subcores
    CH = T // NW
    mesh = plsc.VectorSubcoreMesh(core_axis_name="c", subcore_axis_name="s")

    @functools.partial(
        pl.kernel, mesh=mesh,
        out_type=jax.ShapeDtypeStruct((S, D), jnp.float32),
        scratch_types=[
            pltpu.VMEM((CH, D), jnp.float32),
            pltpu.VMEM((CH,), jnp.int32),
            pltpu.VMEM((CH,), jnp.int32),
            pltpu.SemaphoreType.DMA,
        ],
    )
    def _sc_dispatch(h2_hbm, s1_hbm, s2_hbm, x_hbm, rows, i1, i2, sem):
        wid = lax.axis_index("s") * info.num_cores + lax.axis_index("c")
        base = wid * CH
        pltpu.sync_copy(s1_hbm.at[pl.ds(base, CH)], i1)
        pltpu.sync_copy(s2_hbm.at[pl.ds(base, CH)], i2)
        pltpu.sync_copy(h2_hbm.at[pl.ds(base, CH)], rows)
        pltpu.async_copy(rows, x_hbm.at[i1], sem).wait()
        pltpu.async_copy(rows, x_hbm.at[i2], sem).wait()

    xdisp = _sc_dispatch(h2, s1.reshape(T), s2.reshape(T))

    yexp = pl.pallas_call(
        _group_ffn_kernel,
        grid_spec=pltpu.PrefetchScalarGridSpec(
            num_scalar_prefetch=1,
            grid=(NTILES,),
            in_specs=[
                pl.BlockSpec((TM, D), lambda p, te: (p, 0)),
                pl.BlockSpec((1, D, DFF), lambda p, te: (te[p], 0, 0)),
                pl.BlockSpec((1, 1, DFF), lambda p, te: (te[p], 0, 0)),
                pl.BlockSpec((1, DFF, D), lambda p, te: (te[p], 0, 0)),
                pl.BlockSpec((1, 1, D), lambda p, te: (te[p], 0, 0)),
            ],
            out_specs=pl.BlockSpec((TM, D), lambda p, te: (p, 0)),
        ),
        out_shape=jax.ShapeDtypeStruct((S, D), jnp.float32),
    )(te2d.reshape(NTILES), xdisp, W1, b1.reshape(E, 1, DFF),
      W2, b2.reshape(E, 1, D))

    CH2 = (2 * T) // NW
    slots_all = jnp.concatenate([s1.reshape(T), s2.reshape(T)], axis=0)

    @functools.partial(
        pl.kernel, mesh=mesh,
        out_type=jax.ShapeDtypeStruct((2 * T, D), jnp.float32),
        scratch_types=[
            pltpu.VMEM((CH2,), jnp.int32),
            pltpu.VMEM((CH2, D), jnp.float32),
            pltpu.SemaphoreType.DMA,
        ],
    )
    def _sc_combine_gather(y_hbm, idx_hbm, out_hbm, idxv, buf, sem):
        wid = lax.axis_index("s") * info.num_cores + lax.axis_index("c")
        base = wid * CH2
        pltpu.sync_copy(idx_hbm.at[pl.ds(base, CH2)], idxv)
        pltpu.async_copy(y_hbm.at[idxv], buf, sem).wait()
        pltpu.sync_copy(buf, out_hbm.at[pl.ds(base, CH2)])

    yg = _sc_combine_gather(yexp, slots_all)

    out = pl.pallas_call(
        _combine_kernel,
        grid=(nt,),
        in_specs=[
            pl.BlockSpec((tt, D), lambda i: (i, 0)),
            pl.BlockSpec((tt, D), lambda i: (i, 0)),
            pl.BlockSpec((tt, D), lambda i: (nt + i, 0)),
            pl.BlockSpec((tt, 1), lambda i: (i, 0)),
            pl.BlockSpec((tt, 1), lambda i: (i, 0)),
        ],
        out_specs=pl.BlockSpec((tt, D), lambda i: (i, 0)),
        out_shape=jax.ShapeDtypeStruct((T, D), jnp.float32),
    )(x2, yg, yg, c1, c2)

    return out.reshape(B, T, D)


# speedup vs baseline: 1.5057x; 1.5057x over previous
import functools

import jax
import jax.numpy as jnp
from jax import lax
from jax.experimental import pallas as pl
from jax.experimental.pallas import tpu as pltpu
from jax.experimental.pallas import tpu_sc as plsc


TT = 256
TM = 256


def _qkv_kernel(x_ref, g_ref, w_ref, out_ref):
    xv = x_ref[...]
    h = xv * jax.lax.rsqrt(jnp.mean(xv * xv, axis=-1, keepdims=True) + 1e-6)
    h = (h * g_ref[...]).astype(jnp.bfloat16)
    out_ref[...] = jnp.dot(h, w_ref[...],
                           preferred_element_type=jnp.float32
                           ).astype(jnp.bfloat16)


def _attn_kernel(q_ref, k_ref, v_ref, o_ref, *, n_heads, dk, toff):
    i = pl.program_id(0) + toff
    q = q_ref[...]
    k = k_ref[...]
    v = v_ref[...]
    nq, nk = q.shape[0], k.shape[0]
    row = i * nq + jax.lax.broadcasted_iota(jnp.int32, (nq, nk), 0)
    col = jax.lax.broadcasted_iota(jnp.int32, (nq, nk), 1)
    causal = col <= row
    outs = []
    for h in range(n_heads):
        qh = q[:, h * dk:(h + 1) * dk]
        kh = k[:, h * dk:(h + 1) * dk]
        vh = v[:, h * dk:(h + 1) * dk]
        s = jax.lax.dot_general(qh, kh, (((1,), (1,)), ((), ())),
                                preferred_element_type=jnp.float32)
        s = jnp.where(causal, s, -jnp.inf)
        m = jnp.max(s, axis=-1, keepdims=True)
        p = jnp.exp(s - m)
        l = jnp.sum(p, axis=-1, keepdims=True)
        o = jnp.dot(p.astype(jnp.bfloat16), vh,
                    preferred_element_type=jnp.float32)
        outs.append((o / l).astype(jnp.bfloat16))
    o_ref[...] = jnp.concatenate(outs, axis=1)


def _proj_router_kernel(x_ref, o_ref, w_ref, b_ref, g_ref, wg_ref, bg_ref,
                        wv_ref, bv_ref, n_ref,
                        x2_ref, h2_ref, coef_ref, tm_ref):
    x2 = (x_ref[...]
          + jnp.dot(o_ref[...], w_ref[...],
                    preferred_element_type=jnp.float32)
          + b_ref[...])
    x2_ref[...] = x2
    h2 = x2 * jax.lax.rsqrt(jnp.mean(x2 * x2, axis=-1, keepdims=True) + 1e-6)
    h2 = h2 * g_ref[...]
    h2_ref[...] = h2
    lg = jnp.dot(h2, wg_ref[...], preferred_element_type=jnp.float32) + bg_ref[...]
    lv = jnp.dot(h2, wv_ref[...], preferred_element_type=jnp.float32) + bv_ref[...]
    sp = jnp.maximum(lv, 0.0) + jnp.log(1.0 + jnp.exp(-jnp.abs(lv)))
    logits = lg + n_ref[...] * sp
    m1 = jnp.max(logits, axis=-1, keepdims=True)
    neg = jnp.where(logits == m1, -jnp.inf, logits)
    m2 = jnp.max(neg, axis=-1, keepdims=True)
    tmask = logits >= m2
    z = jnp.where(tmask, jnp.exp(logits - m1), 0.0)
    coef_ref[...] = z / jnp.sum(z, axis=-1, keepdims=True)
    tm_ref[...] = tmask.astype(jnp.float32)


def _offsets_kernel(tm_ref, te_ref, offp_ref, *, n_experts, n_tiles, tile):
    cnt = jnp.sum(tm_ref[...], axis=0, keepdims=True)
    ntile = jnp.ceil(cnt / tile)
    e = n_experts
    mT = (jax.lax.broadcasted_iota(jnp.int32, (e, e), 0)
          < jax.lax.broadcasted_iota(jnp.int32, (e, e), 1)).astype(jnp.float32)
    toff = jnp.dot(ntile, mT, preferred_element_type=jnp.float32)
    offp_ref[...] = toff * tile
    m = (jax.lax.broadcasted_iota(jnp.int32, (e, e), 1)
         < jax.lax.broadcasted_iota(jnp.int32, (e, e), 0)).astype(jnp.float32)
    ntile_b = jnp.broadcast_to(ntile, (e, e))
    toff_s = jnp.sum(m * ntile_b, axis=1, keepdims=True)
    pio = jax.lax.broadcasted_iota(
        jnp.int32, (e, n_tiles), 1).astype(jnp.float32)
    ind = (toff_s <= pio).astype(jnp.int32)
    te = jnp.sum(ind, axis=0, keepdims=True) - 1
    te_ref[...] = jnp.clip(te, 0, n_experts - 1)


def _slots_kernel(tmf_ref, coef_ref, tm_ref, offp_ref,
                  s1_ref, s2_ref, c1_ref, c2_ref, *, n_slots):
    i = pl.program_id(0)
    tt, e = coef_ref.shape
    t_all = tmf_ref.shape[0]
    rowg = i * tt + jax.lax.broadcasted_iota(jnp.int32, (tt, t_all), 0)
    colg = jax.lax.broadcasted_iota(jnp.int32, (tt, t_all), 1)
    ltri = (colg < rowg).astype(jnp.float32)
    rank = jnp.dot(ltri, tmf_ref[...], preferred_element_type=jnp.float32)
    eiota = jax.lax.broadcasted_iota(jnp.int32, (tt, e), 1)
    sel = tm_ref[...] > 0.5
    e1 = jnp.min(jnp.where(sel, eiota, 99), axis=-1, keepdims=True)
    e2 = jnp.max(jnp.where(sel, eiota, -1), axis=-1, keepdims=True)
    oh1 = eiota == e1
    oh2 = eiota == e2
    coef = coef_ref[...]
    offb = jnp.broadcast_to(offp_ref[...], (tt, e))
    c1_ref[...] = jnp.sum(jnp.where(oh1, coef, 0.0), axis=-1, keepdims=True)
    c2_ref[...] = jnp.sum(jnp.where(oh2, coef, 0.0), axis=-1, keepdims=True)
    s1 = jnp.sum(jnp.where(oh1, offb + rank, 0.0), axis=-1, keepdims=True)
    s2 = jnp.sum(jnp.where(oh2, offb + rank, 0.0), axis=-1, keepdims=True)
    s1_ref[...] = jnp.clip(s1.astype(jnp.int32), 0, n_slots - 1)
    s2_ref[...] = jnp.clip(s2.astype(jnp.int32), 0, n_slots - 1)


def _group_ffn_kernel(te_ref, x_ref, w1_ref, b1_ref, w2_ref, b2_ref, y_ref):
    xb = x_ref[...].astype(jnp.bfloat16)
    hm = jnp.maximum(
        jnp.dot(xb, w1_ref[0].astype(jnp.bfloat16),
                preferred_element_type=jnp.float32) + b1_ref[0], 0.0)
    y_ref[...] = (jnp.dot(hm.astype(jnp.bfloat16),
                          w2_ref[0].astype(jnp.bfloat16),
                          preferred_element_type=jnp.float32)
                  + b2_ref[0])


def _combine_kernel(x2_ref, y1_ref, y2_ref, c1_ref, c2_ref, out_ref):
    out_ref[...] = (x2_ref[...]
                    + c1_ref[...] * y1_ref[...]
                    + c2_ref[...] * y2_ref[...])


def kernel(x, noise, g1, g2, Wq, Wk, Wv, Wo, bo, Wg, bg, Wvar, bvar,
           W1, b1, W2, b2):
    B, T, D = x.shape
    H, _, DK = Wq.shape
    E = Wg.shape[1]
    DFF = W1.shape[2]
    HD = H * DK
    tt = min(TT, T)
    nt = T // tt
    K = 2
    NTILES = (T * K) // TM + E
    S = NTILES * TM

    x2d = x.reshape(T, D)
    n2d = noise.reshape(T, E)
    g1r = g1.reshape(1, D)
    g2r = g2.reshape(1, D)
    bor = bo.reshape(1, D)
    bgr = bg.reshape(1, E)
    bvr = bvar.reshape(1, E)
    Wqs = Wq * (1.0 / (DK ** 0.5))
    Wqkv = jnp.concatenate(
        [jnp.transpose(w, (1, 0, 2)).reshape(D, HD) for w in (Wqs, Wk, Wv)],
        axis=1).astype(jnp.bfloat16)
    Wob = Wo.astype(jnp.bfloat16)

    qkv = pl.pallas_call(
        _qkv_kernel,
        grid=(nt,),
        in_specs=[
            pl.BlockSpec((tt, D), lambda i: (i, 0)),
            pl.BlockSpec((1, D), lambda i: (0, 0)),
            pl.BlockSpec((D, 3 * HD), lambda i: (0, 0)),
        ],
        out_specs=pl.BlockSpec((tt, 3 * HD), lambda i: (i, 0)),
        out_shape=jax.ShapeDtypeStruct((T, 3 * HD), jnp.bfloat16),
    )(x2d, g1r, Wqkv)

    nh = nt // 2
    o2a = pl.pallas_call(
        functools.partial(_attn_kernel, n_heads=H, dk=DK, toff=0),
        grid=(nh,),
        in_specs=[
            pl.BlockSpec((tt, HD), lambda i: (i, 0)),
            pl.BlockSpec((T // 2, HD), lambda i: (0, 1)),
            pl.BlockSpec((T // 2, HD), lambda i: (0, 2)),
        ],
        out_specs=pl.BlockSpec((tt, HD), lambda i: (i, 0)),
        out_shape=jax.ShapeDtypeStruct((T // 2, HD), jnp.bfloat16),
    )(qkv, qkv, qkv)
    o2b = pl.pallas_call(
        functools.partial(_attn_kernel, n_heads=H, dk=DK, toff=nh),
        grid=(nt - nh,),
        in_specs=[
            pl.BlockSpec((tt, HD), lambda i: (i + nh, 0)),
            pl.BlockSpec((T, HD), lambda i: (0, 1)),
            pl.BlockSpec((T, HD), lambda i: (0, 2)),
        ],
        out_specs=pl.BlockSpec((tt, HD), lambda i: (i, 0)),
        out_shape=jax.ShapeDtypeStruct((T - T // 2, HD), jnp.bfloat16),
    )(qkv, qkv, qkv)
    o2 = jnp.concatenate([o2a, o2b], axis=0)

    x2, h2, coef, tm = pl.pallas_call(
        _proj_router_kernel,
        grid=(nt,),
        in_specs=[
            pl.BlockSpec((tt, D), lambda i: (i, 0)),
            pl.BlockSpec((tt, HD), lambda i: (i, 0)),
            pl.BlockSpec((D, D), lambda i: (0, 0)),
            pl.BlockSpec((1, D), lambda i: (0, 0)),
            pl.BlockSpec((1, D), lambda i: (0, 0)),
            pl.BlockSpec((D, E), lambda i: (0, 0)),
            pl.BlockSpec((1, E), lambda i: (0, 0)),
            pl.BlockSpec((D, E), lambda i: (0, 0)),
            pl.BlockSpec((1, E), lambda i: (0, 0)),
            pl.BlockSpec((tt, E), lambda i: (i, 0)),
        ],
        out_specs=[
            pl.BlockSpec((tt, D), lambda i: (i, 0)),
            pl.BlockSpec((tt, D), lambda i: (i, 0)),
            pl.BlockSpec((tt, E), lambda i: (i, 0)),
            pl.BlockSpec((tt, E), lambda i: (i, 0)),
        ],
        out_shape=[
            jax.ShapeDtypeStruct((T, D), jnp.float32),
            jax.ShapeDtypeStruct((T, D), jnp.float32),
            jax.ShapeDtypeStruct((T, E), jnp.float32),
            jax.ShapeDtypeStruct((T, E), jnp.float32),
        ],
    )(x2d, o2, Wob, bor, g2r, Wg, bgr, Wvar, bvr, n2d)

    te2d, offp = pl.pallas_call(
        functools.partial(_offsets_kernel, n_experts=E, n_tiles=NTILES,
                          tile=TM),
        grid=(1,),
        in_specs=[pl.BlockSpec((T, E), lambda i: (0, 0))],
        out_specs=[
            pl.BlockSpec((1, NTILES), lambda i: (0, 0)),
            pl.BlockSpec((1, E), lambda i: (0, 0)),
        ],
        out_shape=[
            jax.ShapeDtypeStruct((1, NTILES), jnp.int32),
            jax.ShapeDtypeStruct((1, E), jnp.float32),
        ],
    )(tm)

    s1, s2, c1, c2 = pl.pallas_call(
        functools.partial(_slots_kernel, n_slots=S),
        grid=(nt,),
        in_specs=[
            pl.BlockSpec((T, E), lambda i: (0, 0)),
            pl.BlockSpec((tt, E), lambda i: (i, 0)),
            pl.BlockSpec((tt, E), lambda i: (i, 0)),
            pl.BlockSpec((1, E), lambda i: (0, 0)),
        ],
        out_specs=[
            pl.BlockSpec((tt, 1), lambda i: (i, 0)),
            pl.BlockSpec((tt, 1), lambda i: (i, 0)),
            pl.BlockSpec((tt, 1), lambda i: (i, 0)),
            pl.BlockSpec((tt, 1), lambda i: (i, 0)),
        ],
        out_shape=[
            jax.ShapeDtypeStruct((T, 1), jnp.int32),
            jax.ShapeDtypeStruct((T, 1), jnp.int32),
            jax.ShapeDtypeStruct((T, 1), jnp.float32),
            jax.ShapeDtypeStruct((T, 1), jnp.float32),
        ],
    )(tm, coef, tm, offp)

    info = plsc.get_sparse_core_info()
    NW = info.num_cores * info.num_subcores
    CH = T // NW
    mesh = plsc.VectorSubcoreMesh(core_axis_name="c", subcore_axis_name="s")

    @functools.partial(
        pl.kernel, mesh=mesh,
        out_type=jax.ShapeDtypeStruct((S, D), jnp.float32),
        scratch_types=[
            pltpu.VMEM((CH, D), jnp.float32),
            pltpu.VMEM((CH,), jnp.int32),
            pltpu.VMEM((CH,), jnp.int32),
            pltpu.SemaphoreType.DMA,
        ],
    )
    def _sc_dispatch(h2_hbm, s1_hbm, s2_hbm, x_hbm, rows, i1, i2, sem):
        wid = lax.axis_index("s") * info.num_cores + lax.axis_index("c")
        base = wid * CH
        pltpu.sync_copy(s1_hbm.at[pl.ds(base, CH)], i1)
        pltpu.sync_copy(s2_hbm.at[pl.ds(base, CH)], i2)
        pltpu.sync_copy(h2_hbm.at[pl.ds(base, CH)], rows)
        pltpu.async_copy(rows, x_hbm.at[i1], sem).wait()
        pltpu.async_copy(rows, x_hbm.at[i2], sem).wait()

    xdisp = _sc_dispatch(h2, s1.reshape(T), s2.reshape(T))

    yexp = pl.pallas_call(
        _group_ffn_kernel,
        grid_spec=pltpu.PrefetchScalarGridSpec(
            num_scalar_prefetch=1,
            grid=(NTILES,),
            in_specs=[
                pl.BlockSpec((TM, D), lambda p, te: (p, 0)),
                pl.BlockSpec((1, D, DFF), lambda p, te: (te[p], 0, 0)),
                pl.BlockSpec((1, 1, DFF), lambda p, te: (te[p], 0, 0)),
                pl.BlockSpec((1, DFF, D), lambda p, te: (te[p], 0, 0)),
                pl.BlockSpec((1, 1, D), lambda p, te: (te[p], 0, 0)),
            ],
            out_specs=pl.BlockSpec((TM, D), lambda p, te: (p, 0)),
        ),
        out_shape=jax.ShapeDtypeStruct((S, D), jnp.float32),
    )(te2d.reshape(NTILES), xdisp, W1, b1.reshape(E, 1, DFF),
      W2, b2.reshape(E, 1, D))

    CH2 = (2 * T) // NW
    slots_all = jnp.concatenate([s1.reshape(T), s2.reshape(T)], axis=0)

    @functools.partial(
        pl.kernel, mesh=mesh,
        out_type=jax.ShapeDtypeStruct((2 * T, D), jnp.float32),
        scratch_types=[
            pltpu.VMEM((CH2,), jnp.int32),
            pltpu.VMEM((CH2, D), jnp.float32),
            pltpu.SemaphoreType.DMA,
        ],
    )
    def _sc_combine_gather(y_hbm, idx_hbm, out_hbm, idxv, buf, sem):
        wid = lax.axis_index("s") * info.num_cores + lax.axis_index("c")
        base = wid * CH2
        pltpu.sync_copy(idx_hbm.at[pl.ds(base, CH2)], idxv)
        pltpu.async_copy(y_hbm.at[idxv], buf, sem).wait()
        pltpu.sync_copy(buf, out_hbm.at[pl.ds(base, CH2)])

    yg = _sc_combine_gather(yexp, slots_all)

    out = pl.pallas_call(
        _combine_kernel,
        grid=(nt,),
        in_specs=[
            pl.BlockSpec((tt, D), lambda i: (i, 0)),
            pl.BlockSpec((tt, D), lambda i: (i, 0)),
            pl.BlockSpec((tt, D), lambda i: (nt + i, 0)),
            pl.BlockSpec((tt, 1), lambda i: (i, 0)),
            pl.BlockSpec((tt, 1), lambda i: (i, 0)),
        ],
        out_specs=pl.BlockSpec((tt, D), lambda i: (i, 0)),
        out_shape=jax.ShapeDtypeStruct((T, D), jnp.float32),
    )(x2, yg, yg, c1, c2)

    return out.reshape(B, T, D)


# speedup vs baseline: 1.5183x; 1.0084x over previous
import functools

import jax
import jax.numpy as jnp
from jax import lax
from jax.experimental import pallas as pl
from jax.experimental.pallas import tpu as pltpu
from jax.experimental.pallas import tpu_sc as plsc


TT = 256
TM = 256


def _qkv_kernel(x_ref, g_ref, w_ref, out_ref):
    xv = x_ref[...]
    h = xv * jax.lax.rsqrt(jnp.mean(xv * xv, axis=-1, keepdims=True) + 1e-6)
    h = (h * g_ref[...]).astype(jnp.bfloat16)
    out_ref[...] = jnp.dot(h, w_ref[...],
                           preferred_element_type=jnp.float32
                           ).astype(jnp.bfloat16)


def _attn_kernel(q_ref, k_ref, v_ref, o_ref, *, n_heads, dk, toff):
    i = pl.program_id(0) + toff
    q = q_ref[...]
    k = k_ref[...]
    v = v_ref[...]
    nq, nk = q.shape[0], k.shape[0]
    row = i * nq + jax.lax.broadcasted_iota(jnp.int32, (nq, nk), 0)
    col = jax.lax.broadcasted_iota(jnp.int32, (nq, nk), 1)
    causal = col <= row
    outs = []
    for h in range(n_heads):
        qh = q[:, h * dk:(h + 1) * dk]
        kh = k[:, h * dk:(h + 1) * dk]
        vh = v[:, h * dk:(h + 1) * dk]
        s = jax.lax.dot_general(qh, kh, (((1,), (1,)), ((), ())),
                                preferred_element_type=jnp.float32)
        s = jnp.where(causal, s, -jnp.inf)
        m = jnp.max(s, axis=-1, keepdims=True)
        p = jnp.exp(s - m)
        l = jnp.sum(p, axis=-1, keepdims=True)
        o = jnp.dot(p.astype(jnp.bfloat16), vh,
                    preferred_element_type=jnp.float32)
        outs.append((o / l).astype(jnp.bfloat16))
    o_ref[...] = jnp.concatenate(outs, axis=1)


def _proj_router_kernel(x_ref, o_ref, w_ref, b_ref, g_ref, wg_ref, bg_ref,
                        wv_ref, bv_ref, n_ref,
                        x2_ref, h2_ref, coef_ref, tm_ref):
    x2 = (x_ref[...]
          + jnp.dot(o_ref[...], w_ref[...],
                    preferred_element_type=jnp.float32)
          + b_ref[...])
    x2_ref[...] = x2
    h2 = x2 * jax.lax.rsqrt(jnp.mean(x2 * x2, axis=-1, keepdims=True) + 1e-6)
    h2 = h2 * g_ref[...]
    h2_ref[...] = h2
    lg = jnp.dot(h2, wg_ref[...], preferred_element_type=jnp.float32) + bg_ref[...]
    lv = jnp.dot(h2, wv_ref[...], preferred_element_type=jnp.float32) + bv_ref[...]
    sp = jnp.maximum(lv, 0.0) + jnp.log(1.0 + jnp.exp(-jnp.abs(lv)))
    logits = lg + n_ref[...] * sp
    m1 = jnp.max(logits, axis=-1, keepdims=True)
    neg = jnp.where(logits == m1, -jnp.inf, logits)
    m2 = jnp.max(neg, axis=-1, keepdims=True)
    tmask = logits >= m2
    z = jnp.where(tmask, jnp.exp(logits - m1), 0.0)
    coef_ref[...] = z / jnp.sum(z, axis=-1, keepdims=True)
    tm_ref[...] = tmask.astype(jnp.float32)


def _offsets_kernel(tm_ref, te_ref, offp_ref, *, n_experts, n_tiles, tile):
    cnt = jnp.sum(tm_ref[...], axis=0, keepdims=True)
    ntile = jnp.ceil(cnt / tile)
    e = n_experts
    mT = (jax.lax.broadcasted_iota(jnp.int32, (e, e), 0)
          < jax.lax.broadcasted_iota(jnp.int32, (e, e), 1)).astype(jnp.float32)
    toff = jnp.dot(ntile, mT, preferred_element_type=jnp.float32)
    offp_ref[...] = toff * tile
    m = (jax.lax.broadcasted_iota(jnp.int32, (e, e), 1)
         < jax.lax.broadcasted_iota(jnp.int32, (e, e), 0)).astype(jnp.float32)
    ntile_b = jnp.broadcast_to(ntile, (e, e))
    toff_s = jnp.sum(m * ntile_b, axis=1, keepdims=True)
    pio = jax.lax.broadcasted_iota(
        jnp.int32, (e, n_tiles), 1).astype(jnp.float32)
    ind = (toff_s <= pio).astype(jnp.int32)
    te = jnp.sum(ind, axis=0, keepdims=True) - 1
    te_ref[...] = jnp.clip(te, 0, n_experts - 1)


def _slots_kernel(tmf_ref, coef_ref, tm_ref, offp_ref,
                  s1_ref, s2_ref, c1_ref, c2_ref, *, n_slots):
    i = pl.program_id(0)
    tt, e = coef_ref.shape
    t_all = tmf_ref.shape[0]
    rowg = i * tt + jax.lax.broadcasted_iota(jnp.int32, (tt, t_all), 0)
    colg = jax.lax.broadcasted_iota(jnp.int32, (tt, t_all), 1)
    ltri = (colg < rowg).astype(jnp.float32)
    rank = jnp.dot(ltri, tmf_ref[...], preferred_element_type=jnp.float32)
    eiota = jax.lax.broadcasted_iota(jnp.int32, (tt, e), 1)
    sel = tm_ref[...] > 0.5
    e1 = jnp.min(jnp.where(sel, eiota, 99), axis=-1, keepdims=True)
    e2 = jnp.max(jnp.where(sel, eiota, -1), axis=-1, keepdims=True)
    oh1 = eiota == e1
    oh2 = eiota == e2
    coef = coef_ref[...]
    offb = jnp.broadcast_to(offp_ref[...], (tt, e))
    c1_ref[...] = jnp.sum(jnp.where(oh1, coef, 0.0), axis=-1, keepdims=True)
    c2_ref[...] = jnp.sum(jnp.where(oh2, coef, 0.0), axis=-1, keepdims=True)
    s1 = jnp.sum(jnp.where(oh1, offb + rank, 0.0), axis=-1, keepdims=True)
    s2 = jnp.sum(jnp.where(oh2, offb + rank, 0.0), axis=-1, keepdims=True)
    s1_ref[...] = jnp.clip(s1.astype(jnp.int32), 0, n_slots - 1)
    s2_ref[...] = jnp.clip(s2.astype(jnp.int32), 0, n_slots - 1)


def _group_ffn_kernel(te_ref, x_ref, w1_ref, b1_ref, w2_ref, b2_ref, y_ref):
    xb = x_ref[...].astype(jnp.bfloat16)
    hm = jnp.maximum(
        jnp.dot(xb, w1_ref[0].astype(jnp.bfloat16),
                preferred_element_type=jnp.float32) + b1_ref[0], 0.0)
    y_ref[...] = (jnp.dot(hm.astype(jnp.bfloat16),
                          w2_ref[0].astype(jnp.bfloat16),
                          preferred_element_type=jnp.float32)
                  + b2_ref[0])


def _combine_kernel(x2_ref, y1_ref, y2_ref, c1_ref, c2_ref, out_ref):
    out_ref[...] = (x2_ref[...]
                    + c1_ref[...] * y1_ref[...]
                    + c2_ref[...] * y2_ref[...])


def kernel(x, noise, g1, g2, Wq, Wk, Wv, Wo, bo, Wg, bg, Wvar, bvar,
           W1, b1, W2, b2):
    B, T, D = x.shape
    H, _, DK = Wq.shape
    E = Wg.shape[1]
    DFF = W1.shape[2]
    HD = H * DK
    tt = min(TT, T)
    nt = T // tt
    K = 2
    NTILES = (T * K) // TM + E
    S = NTILES * TM

    x2d = x.reshape(T, D)
    n2d = noise.reshape(T, E)
    g1r = g1.reshape(1, D)
    g2r = g2.reshape(1, D)
    bor = bo.reshape(1, D)
    bgr = bg.reshape(1, E)
    bvr = bvar.reshape(1, E)
    Wqs = Wq * (1.0 / (DK ** 0.5))
    Wqkv = jnp.concatenate(
        [jnp.transpose(w, (1, 0, 2)).reshape(D, HD) for w in (Wqs, Wk, Wv)],
        axis=1).astype(jnp.bfloat16)
    Wob = Wo.astype(jnp.bfloat16)

    qkv = pl.pallas_call(
        _qkv_kernel,
        grid=(nt,),
        in_specs=[
            pl.BlockSpec((tt, D), lambda i: (i, 0)),
            pl.BlockSpec((1, D), lambda i: (0, 0)),
            pl.BlockSpec((D, 3 * HD), lambda i: (0, 0)),
        ],
        out_specs=pl.BlockSpec((tt, 3 * HD), lambda i: (i, 0)),
        out_shape=jax.ShapeDtypeStruct((T, 3 * HD), jnp.bfloat16),
    )(x2d, g1r, Wqkv)

    nsplit = 4
    tiles_per = nt // nsplit
    parts = []
    for c in range(nsplit):
        kw = (c + 1) * tiles_per * tt
        toff = c * tiles_per
        parts.append(pl.pallas_call(
            functools.partial(_attn_kernel, n_heads=H, dk=DK, toff=toff),
            grid=(tiles_per,),
            in_specs=[
                pl.BlockSpec((tt, HD), lambda i, toff=toff: (i + toff, 0)),
                pl.BlockSpec((kw, HD), lambda i: (0, 1)),
                pl.BlockSpec((kw, HD), lambda i: (0, 2)),
            ],
            out_specs=pl.BlockSpec((tt, HD), lambda i: (i, 0)),
            out_shape=jax.ShapeDtypeStruct((tiles_per * tt, HD),
                                           jnp.bfloat16),
        )(qkv, qkv, qkv))
    o2 = jnp.concatenate(parts, axis=0)

    x2, h2, coef, tm = pl.pallas_call(
        _proj_router_kernel,
        grid=(nt,),
        in_specs=[
            pl.BlockSpec((tt, D), lambda i: (i, 0)),
            pl.BlockSpec((tt, HD), lambda i: (i, 0)),
            pl.BlockSpec((D, D), lambda i: (0, 0)),
            pl.BlockSpec((1, D), lambda i: (0, 0)),
            pl.BlockSpec((1, D), lambda i: (0, 0)),
            pl.BlockSpec((D, E), lambda i: (0, 0)),
            pl.BlockSpec((1, E), lambda i: (0, 0)),
            pl.BlockSpec((D, E), lambda i: (0, 0)),
            pl.BlockSpec((1, E), lambda i: (0, 0)),
            pl.BlockSpec((tt, E), lambda i: (i, 0)),
        ],
        out_specs=[
            pl.BlockSpec((tt, D), lambda i: (i, 0)),
            pl.BlockSpec((tt, D), lambda i: (i, 0)),
            pl.BlockSpec((tt, E), lambda i: (i, 0)),
            pl.BlockSpec((tt, E), lambda i: (i, 0)),
        ],
        out_shape=[
            jax.ShapeDtypeStruct((T, D), jnp.float32),
            jax.ShapeDtypeStruct((T, D), jnp.float32),
            jax.ShapeDtypeStruct((T, E), jnp.float32),
            jax.ShapeDtypeStruct((T, E), jnp.float32),
        ],
    )(x2d, o2, Wob, bor, g2r, Wg, bgr, Wvar, bvr, n2d)

    te2d, offp = pl.pallas_call(
        functools.partial(_offsets_kernel, n_experts=E, n_tiles=NTILES,
                          tile=TM),
        grid=(1,),
        in_specs=[pl.BlockSpec((T, E), lambda i: (0, 0))],
        out_specs=[
            pl.BlockSpec((1, NTILES), lambda i: (0, 0)),
            pl.BlockSpec((1, E), lambda i: (0, 0)),
        ],
        out_shape=[
            jax.ShapeDtypeStruct((1, NTILES), jnp.int32),
            jax.ShapeDtypeStruct((1, E), jnp.float32),
        ],
    )(tm)

    s1, s2, c1, c2 = pl.pallas_call(
        functools.partial(_slots_kernel, n_slots=S),
        grid=(nt,),
        in_specs=[
            pl.BlockSpec((T, E), lambda i: (0, 0)),
            pl.BlockSpec((tt, E), lambda i: (i, 0)),
            pl.BlockSpec((tt, E), lambda i: (i, 0)),
            pl.BlockSpec((1, E), lambda i: (0, 0)),
        ],
        out_specs=[
            pl.BlockSpec((tt, 1), lambda i: (i, 0)),
            pl.BlockSpec((tt, 1), lambda i: (i, 0)),
            pl.BlockSpec((tt, 1), lambda i: (i, 0)),
            pl.BlockSpec((tt, 1), lambda i: (i, 0)),
        ],
        out_shape=[
            jax.ShapeDtypeStruct((T, 1), jnp.int32),
            jax.ShapeDtypeStruct((T, 1), jnp.int32),
            jax.ShapeDtypeStruct((T, 1), jnp.float32),
            jax.ShapeDtypeStruct((T, 1), jnp.float32),
        ],
    )(tm, coef, tm, offp)

    info = plsc.get_sparse_core_info()
    NW = info.num_cores * info.num_subcores
    CH = T // NW
    mesh = plsc.VectorSubcoreMesh(core_axis_name="c", subcore_axis_name="s")

    @functools.partial(
        pl.kernel, mesh=mesh,
        out_type=jax.ShapeDtypeStruct((S, D), jnp.float32),
        scratch_types=[
            pltpu.VMEM((CH, D), jnp.float32),
            pltpu.VMEM((CH,), jnp.int32),
            pltpu.VMEM((CH,), jnp.int32),
            pltpu.SemaphoreType.DMA,
        ],
    )
    def _sc_dispatch(h2_hbm, s1_hbm, s2_hbm, x_hbm, rows, i1, i2, sem):
        wid = lax.axis_index("s") * info.num_cores + lax.axis_index("c")
        base = wid * CH
        pltpu.sync_copy(s1_hbm.at[pl.ds(base, CH)], i1)
        pltpu.sync_copy(s2_hbm.at[pl.ds(base, CH)], i2)
        pltpu.sync_copy(h2_hbm.at[pl.ds(base, CH)], rows)
        pltpu.async_copy(rows, x_hbm.at[i1], sem).wait()
        pltpu.async_copy(rows, x_hbm.at[i2], sem).wait()

    xdisp = _sc_dispatch(h2, s1.reshape(T), s2.reshape(T))

    yexp = pl.pallas_call(
        _group_ffn_kernel,
        grid_spec=pltpu.PrefetchScalarGridSpec(
            num_scalar_prefetch=1,
            grid=(NTILES,),
            in_specs=[
                pl.BlockSpec((TM, D), lambda p, te: (p, 0)),
                pl.BlockSpec((1, D, DFF), lambda p, te: (te[p], 0, 0)),
                pl.BlockSpec((1, 1, DFF), lambda p, te: (te[p], 0, 0)),
                pl.BlockSpec((1, DFF, D), lambda p, te: (te[p], 0, 0)),
                pl.BlockSpec((1, 1, D), lambda p, te: (te[p], 0, 0)),
            ],
            out_specs=pl.BlockSpec((TM, D), lambda p, te: (p, 0)),
        ),
        out_shape=jax.ShapeDtypeStruct((S, D), jnp.float32),
    )(te2d.reshape(NTILES), xdisp, W1, b1.reshape(E, 1, DFF),
      W2, b2.reshape(E, 1, D))

    CH2 = (2 * T) // NW
    slots_all = jnp.concatenate([s1.reshape(T), s2.reshape(T)], axis=0)

    @functools.partial(
        pl.kernel, mesh=mesh,
        out_type=jax.ShapeDtypeStruct((2 * T, D), jnp.float32),
        scratch_types=[
            pltpu.VMEM((CH2,), jnp.int32),
            pltpu.VMEM((CH2, D), jnp.float32),
            pltpu.SemaphoreType.DMA,
        ],
    )
    def _sc_combine_gather(y_hbm, idx_hbm, out_hbm, idxv, buf, sem):
        wid = lax.axis_index("s") * info.num_cores + lax.axis_index("c")
        base = wid * CH2
        pltpu.sync_copy(idx_hbm.at[pl.ds(base, CH2)], idxv)
        pltpu.async_copy(y_hbm.at[idxv], buf, sem).wait()
        pltpu.sync_copy(buf, out_hbm.at[pl.ds(base, CH2)])

    yg = _sc_combine_gather(yexp, slots_all)

    out = pl.pallas_call(
        _combine_kernel,
        grid=(nt,),
        in_specs=[
            pl.BlockSpec((tt, D), lambda i: (i, 0)),
            pl.BlockSpec((tt, D), lambda i: (i, 0)),
            pl.BlockSpec((tt, D), lambda i: (nt + i, 0)),
            pl.BlockSpec((tt, 1), lambda i: (i, 0)),
            pl.BlockSpec((tt, 1), lambda i: (i, 0)),
        ],
        out_specs=pl.BlockSpec((tt, D), lambda i: (i, 0)),
        out_shape=jax.ShapeDtypeStruct((T, D), jnp.float32),
    )(x2, yg, yg, c1, c2)

    return out.reshape(B, T, D)
